# baseline (speedup 1.0000x reference)
import functools
import jax
import jax.numpy as jnp
from jax import lax
from jax.experimental import pallas as pl
from jax.experimental.pallas import tpu as pltpu
from jax.experimental.pallas import tpu_sc as plsc

N = 100000
D = 256
NC = 2
NS = 16
NW = NC * NS
ROWS_W = N // NW
C = 125
CHUNKS = ROWS_W // C

_mesh = plsc.VectorSubcoreMesh(core_axis_name="c", subcore_axis_name="s")


@functools.partial(
    pl.kernel,
    out_type=jax.ShapeDtypeStruct((N, 2, D), jnp.float32),
    mesh=_mesh,
    scratch_types=[
        pltpu.VMEM((C, 1, D), jnp.float32),
        pltpu.VMEM((C, 1, D), jnp.float32),
    ],
)
def _sc_concat(s_hbm, d_hbm, o_hbm, bs, bd):
    wid = lax.axis_index("s") * NC + lax.axis_index("c")
    base = wid * ROWS_W

    def body(i, carry):
        r0 = base + i * C
        rows = pl.ds(r0, C)
        pltpu.sync_copy(s_hbm.at[rows, :, :], bs)
        pltpu.sync_copy(d_hbm.at[rows, :, :], bd)
        pltpu.sync_copy(bs, o_hbm.at[rows, pl.ds(0, 1), :])
        pltpu.sync_copy(bd, o_hbm.at[rows, pl.ds(1, 1), :])
        return carry

    lax.fori_loop(0, CHUNKS, body, 0)


def kernel(static_emb, dynamic_emb):
    out3 = _sc_concat(
        static_emb.reshape(N, 1, D), dynamic_emb.reshape(N, 1, D)
    )
    return out3.reshape(N, 2 * D)


# speedup vs baseline: 1.0377x; 1.0377x over previous
import functools
import jax
import jax.numpy as jnp
from jax import lax
from jax.experimental import pallas as pl
from jax.experimental.pallas import tpu as pltpu
from jax.experimental.pallas import tpu_sc as plsc

N = 100000
D = 256
NC = 2
NS = 16
NW = NC * NS
ROWS_W = N // NW
C = 125
CHUNKS = ROWS_W // C

_mesh = plsc.VectorSubcoreMesh(core_axis_name="c", subcore_axis_name="s")


@functools.partial(
    pl.kernel,
    out_type=jax.ShapeDtypeStruct((N, 2, D), jnp.float32),
    mesh=_mesh,
    scratch_types=[
        pltpu.VMEM((2, C, 1, D), jnp.float32),
        pltpu.VMEM((2, C, 1, D), jnp.float32),
        pltpu.SemaphoreType.DMA((2,)),
        pltpu.SemaphoreType.DMA((2,)),
        pltpu.SemaphoreType.DMA((2,)),
        pltpu.SemaphoreType.DMA((2,)),
    ],
)
def _sc_concat(s_hbm, d_hbm, o_hbm, bs, bd, gs_sem, gd_sem, ss_sem, sd_sem):
    wid = lax.axis_index("s") * NC + lax.axis_index("c")
    base = wid * ROWS_W

    def gathers(i, b):
        rows = pl.ds(base + i * C, C)
        return (
            pltpu.make_async_copy(s_hbm.at[rows, :, :], bs.at[b], gs_sem.at[b]),
            pltpu.make_async_copy(d_hbm.at[rows, :, :], bd.at[b], gd_sem.at[b]),
        )

    def scatters(i, b):
        rows = pl.ds(base + i * C, C)
        return (
            pltpu.make_async_copy(bs.at[b], o_hbm.at[rows, pl.ds(0, 1), :], ss_sem.at[b]),
            pltpu.make_async_copy(bd.at[b], o_hbm.at[rows, pl.ds(1, 1), :], sd_sem.at[b]),
        )

    for i in range(CHUNKS + 1):
        b = i % 2
        pb = (i - 1) % 2
        if i < CHUNKS:
            if i >= 2:
                c1, c2 = scatters(i - 2, b)
                c1.wait()
                c2.wait()
            g1, g2 = gathers(i, b)
            g1.start()
            g2.start()
        if i >= 1:
            g1, g2 = gathers(i - 1, pb)
            g1.wait()
            g2.wait()
            c1, c2 = scatters(i - 1, pb)
            c1.start()
            c2.start()
    for i in (CHUNKS - 2, CHUNKS - 1):
        c1, c2 = scatters(i, i % 2)
        c1.wait()
        c2.wait()


def kernel(static_emb, dynamic_emb):
    out3 = _sc_concat(
        static_emb.reshape(N, 1, D), dynamic_emb.reshape(N, 1, D)
    )
    return out3.reshape(N, 2 * D)


# speedup vs baseline: 5.3166x; 5.1234x over previous
import functools
import jax
import jax.numpy as jnp
from jax import lax
from jax.experimental import pallas as pl
from jax.experimental.pallas import tpu as pltpu
from jax.experimental.pallas import tpu_sc as plsc

N = 100000
D = 256
NC = 2
NS = 16
NW = NC * NS
G = N // 8
GW = -(-G // NW)
ROWS_W = GW * 8
C = 120
FULL_CHUNKS = ROWS_W // C
TAIL = ROWS_W - FULL_CHUNKS * C

_mesh = plsc.VectorSubcoreMesh(core_axis_name="c", subcore_axis_name="s")


@functools.partial(
    pl.kernel,
    out_type=jax.ShapeDtypeStruct((N, 2 * D), jnp.float32),
    mesh=_mesh,
    scratch_types=[
        pltpu.VMEM((2, C, D), jnp.float32),
        pltpu.VMEM((2, C, D), jnp.float32),
        pltpu.SemaphoreType.DMA((2,)),
        pltpu.SemaphoreType.DMA((2,)),
        pltpu.SemaphoreType.DMA((2,)),
        pltpu.SemaphoreType.DMA((2,)),
    ],
)
def _sc_concat(s_hbm, d_hbm, o_hbm, bs, bd, gs_sem, gd_sem, ss_sem, sd_sem):
    wid = lax.axis_index("s") * NC + lax.axis_index("c")
    base = jnp.minimum(wid * ROWS_W, N - ROWS_W)
    base = pl.multiple_of(base, 8)

    def sizes(i):
        return (i * C, C if i < FULL_CHUNKS else TAIL)

    def gathers(i, b):
        off, sz = sizes(i)
        rows = pl.ds(base + off, sz)
        return (
            pltpu.make_async_copy(s_hbm.at[rows, :], bs.at[b, pl.ds(0, sz), :], gs_sem.at[b]),
            pltpu.make_async_copy(d_hbm.at[rows, :], bd.at[b, pl.ds(0, sz), :], gd_sem.at[b]),
        )

    def scatters(i, b):
        off, sz = sizes(i)
        rows = pl.ds(base + off, sz)
        return (
            pltpu.make_async_copy(bs.at[b, pl.ds(0, sz), :], o_hbm.at[rows, pl.ds(0, D)], ss_sem.at[b]),
            pltpu.make_async_copy(bd.at[b, pl.ds(0, sz), :], o_hbm.at[rows, pl.ds(D, D)], sd_sem.at[b]),
        )

    NCHUNKS = FULL_CHUNKS + 1
    for i in range(NCHUNKS + 1):
        b = i % 2
        pb = (i - 1) % 2
        if i < NCHUNKS:
            if i >= 2:
                c1, c2 = scatters(i - 2, b)
                c1.wait()
                c2.wait()
            g1, g2 = gathers(i, b)
            g1.start()
            g2.start()
        if i >= 1:
            g1, g2 = gathers(i - 1, pb)
            g1.wait()
            g2.wait()
            c1, c2 = scatters(i - 1, pb)
            c1.start()
            c2.start()
    for i in (NCHUNKS - 2, NCHUNKS - 1):
        c1, c2 = scatters(i, i % 2)
        c1.wait()
        c2.wait()


def kernel(static_emb, dynamic_emb):
    return _sc_concat(static_emb, dynamic_emb)


# speedup vs baseline: 5.3521x; 1.0067x over previous
import functools
import jax
import jax.numpy as jnp
from jax import lax
from jax.experimental import pallas as pl
from jax.experimental.pallas import tpu as pltpu
from jax.experimental.pallas import tpu_sc as plsc

N = 100000
D = 256
NC = 2
NS = 16
NW = NC * NS
ROWS_W = -(-(N // 8) // NW) * 8
C = 240
FULL_CHUNKS = ROWS_W // C
TAIL = ROWS_W - FULL_CHUNKS * C
NCH = FULL_CHUNKS + (1 if TAIL else 0)

_ITEMS = [(c, w) for c in range(NCH) for w in (0, 1)]

_mesh = plsc.VectorSubcoreMesh(core_axis_name="c", subcore_axis_name="s")


@functools.partial(
    pl.kernel,
    out_type=jax.ShapeDtypeStruct((N, 2 * D), jnp.float32),
    mesh=_mesh,
    scratch_types=[
        pltpu.VMEM((2, C, D), jnp.float32),
        pltpu.SemaphoreType.DMA((2,)),
        pltpu.SemaphoreType.DMA((2,)),
    ],
)
def _sc_concat(s_hbm, d_hbm, o_hbm, buf, g_sem, s_sem):
    wid = lax.axis_index("s") * NC + lax.axis_index("c")
    base = jnp.minimum(wid * ROWS_W, N - ROWS_W)
    base = pl.multiple_of(base, 8)

    def gather(item, b):
        c, w = item
        sz = C if c < FULL_CHUNKS else TAIL
        rows = pl.ds(base + c * C, sz)
        src = (s_hbm, d_hbm)[w]
        return pltpu.make_async_copy(
            src.at[rows, :], buf.at[b, pl.ds(0, sz), :], g_sem.at[b]
        )

    def scatter(item, b):
        c, w = item
        sz = C if c < FULL_CHUNKS else TAIL
        rows = pl.ds(base + c * C, sz)
        return pltpu.make_async_copy(
            buf.at[b, pl.ds(0, sz), :], o_hbm.at[rows, pl.ds(w * D, D)], s_sem.at[b]
        )

    n = len(_ITEMS)
    for i in range(n + 1):
        b = i % 2
        pb = (i - 1) % 2
        if i < n:
            if i >= 2:
                scatter(_ITEMS[i - 2], b).wait()
            gather(_ITEMS[i], b).start()
        if i >= 1:
            gather(_ITEMS[i - 1], pb).wait()
            scatter(_ITEMS[i - 1], pb).start()
    for i in (n - 2, n - 1):
        scatter(_ITEMS[i], i % 2).wait()


def kernel(static_emb, dynamic_emb):
    return _sc_concat(static_emb, dynamic_emb)
